# baseline (speedup 1.0000x reference)
import functools

import jax
import jax.numpy as jnp
from jax import lax
from jax.experimental import pallas as pl
from jax.experimental.pallas import tpu as pltpu
from jax.experimental.pallas import tpu_sc as plsc

N = 10000
NT = 3 * N
E1 = 32 * N
ER = 3 * E1
D = 128
H = 64
B = 4096
WD = 16

NC, NS = 2, 16
NW = NC * NS
EPW = ER // NW
CK = 120
NCHUNK = EPW // CK
RPT_A = 1880
RPT_B = NT - 15 * RPT_A

EPW_C = E1 // NW
CKC = 80
NCHUNK_C = EPW_C // CKC
MG = 5
NGC = NCHUNK_C // MG
CRT_A = 632
CRT_B = N - 15 * CRT_A

_mesh = plsc.VectorSubcoreMesh(core_axis_name="c", subcore_axis_name="s")

DG = 10


@functools.partial(
    pl.kernel,
    out_type=jax.ShapeDtypeStruct((NC, NT, WD), jnp.float32),
    mesh=_mesh,
    compiler_params=pltpu.CompilerParams(use_tc_tiling_on_sc=False),
    scratch_types=[
        pltpu.VMEM((NCHUNK, CK), jnp.int32),
        pltpu.VMEM((CK, WD), jnp.float32),
        pltpu.VMEM((RPT_A, WD), jnp.float32),
        pltpu.VMEM_SHARED((NT, WD), jnp.float32),
        pltpu.SemaphoreType.DMA,
    ],
)
def _deg_sc(dst_hbm, out_hbm, didx_v, ones_v, zrow_v, acc_sh, ssem):
    cid = lax.axis_index("c")
    sid = lax.axis_index("s")
    wid = sid * NC + cid
    r0 = sid * RPT_A
    ones_v[...] = jnp.ones_like(ones_v)
    zrow_v[...] = jnp.zeros_like(zrow_v)

    @pl.when(sid < NS - 1)
    def _():
        pltpu.sync_copy(zrow_v, acc_sh.at[pl.ds(r0, RPT_A)])

    @pl.when(sid == NS - 1)
    def _():
        pltpu.sync_copy(zrow_v.at[pl.ds(0, RPT_B)],
                        acc_sh.at[pl.ds(r0, RPT_B)])

    pltpu.sync_copy(dst_hbm.at[wid], didx_v)
    plsc.subcore_barrier()

    def body(g, carry):
        for j in range(DG):
            pltpu.async_copy(ones_v, acc_sh.at[didx_v.at[g * DG + j]],
                             ssem, add=True)
        for j in range(DG):
            pltpu.make_async_copy(ones_v, acc_sh.at[pl.ds(0, CK)],
                                  ssem).wait()
        return carry

    lax.fori_loop(0, NCHUNK // DG, body, 0)
    plsc.subcore_barrier()

    @pl.when(sid < NS - 1)
    def _():
        pltpu.sync_copy(acc_sh.at[pl.ds(r0, RPT_A)],
                        out_hbm.at[cid, pl.ds(r0, RPT_A)])

    @pl.when(sid == NS - 1)
    def _():
        pltpu.sync_copy(acc_sh.at[pl.ds(r0, RPT_B)],
                        out_hbm.at[cid, pl.ds(r0, RPT_B)])

@functools.partial(
    pl.kernel,
    out_type=jax.ShapeDtypeStruct((NC, NT, H), jnp.float32),
    mesh=_mesh,
    compiler_params=pltpu.CompilerParams(use_tc_tiling_on_sc=False),
    scratch_types=[
        pltpu.VMEM((NCHUNK_C, CKC), jnp.int32),
        pltpu.VMEM((NCHUNK_C, CKC), jnp.int32),
        pltpu.VMEM((MG, CKC, H), jnp.float32),
        pltpu.VMEM((MG, CKC, H), jnp.float32),
        pltpu.VMEM_SHARED((N, H), jnp.float32),
        pltpu.SemaphoreType.DMA,
        pltpu.SemaphoreType.DMA,
        pltpu.SemaphoreType.DMA,
        pltpu.SemaphoreType.DMA,
    ],
)
def _conv_sc(src_hbm, dst_hbm, h_hbm, z_hbm, out_hbm,
             sidx_v, didx_v, rows_a, rows_b, acc_sh,
             gs_a, gs_b, ss_a, ss_b):
    cid = lax.axis_index("c")
    sid = lax.axis_index("s")
    wid = sid * NC + cid
    r0 = sid * CRT_A

    def fire_g(g, buf, sem):
        for j in range(MG):
            pltpu.async_copy(h_hbm.at[sidx_v.at[g * MG + j]], buf.at[j], sem)

    def drain_g(buf, sem):
        for j in range(MG):
            pltpu.make_async_copy(
                h_hbm.at[pl.ds(0, CKC)], buf.at[j], sem).wait()

    def fire_s(g, buf, sem):
        for j in range(MG):
            pltpu.async_copy(
                buf.at[j], acc_sh.at[didx_v.at[g * MG + j]], sem, add=True)

    def drain_s(buf, sem):
        for j in range(MG):
            pltpu.make_async_copy(
                buf.at[j], acc_sh.at[pl.ds(0, CKC)], sem).wait()

    for t in range(3):
        tb = t * N
        @pl.when((cid == 0) & (sid < NS - 1))
        def _():
            pltpu.sync_copy(h_hbm.at[pl.ds(tb + r0, CRT_A)],
                            acc_sh.at[pl.ds(r0, CRT_A)])

        @pl.when((cid == 0) & (sid == NS - 1))
        def _():
            pltpu.sync_copy(h_hbm.at[pl.ds(tb + r0, CRT_B)],
                            acc_sh.at[pl.ds(r0, CRT_B)])

        @pl.when((cid == 1) & (sid < NS - 1))
        def _():
            pltpu.sync_copy(z_hbm, acc_sh.at[pl.ds(r0, CRT_A)])

        @pl.when((cid == 1) & (sid == NS - 1))
        def _():
            pltpu.sync_copy(z_hbm.at[pl.ds(0, CRT_B)],
                            acc_sh.at[pl.ds(r0, CRT_B)])

        pltpu.sync_copy(src_hbm.at[t, wid], sidx_v)
        pltpu.sync_copy(dst_hbm.at[t, wid], didx_v)
        plsc.subcore_barrier()

        fire_g(0, rows_a, gs_a)
        drain_g(rows_a, gs_a)
        fire_g(1, rows_b, gs_b)
        fire_s(0, rows_a, ss_a)

        def body(i, carry):
            g = 2 * i + 1
            drain_g(rows_b, gs_b)
            drain_s(rows_a, ss_a)
            fire_g(g + 1, rows_a, gs_a)
            fire_s(g, rows_b, ss_b)
            drain_g(rows_a, gs_a)
            drain_s(rows_b, ss_b)
            fire_g(g + 2, rows_b, gs_b)
            fire_s(g + 1, rows_a, ss_a)
            return carry

        lax.fori_loop(0, (NGC - 3) // 2, body, 0)
        drain_g(rows_b, gs_b)
        drain_s(rows_a, ss_a)
        fire_g(NGC - 1, rows_a, gs_a)
        fire_s(NGC - 2, rows_b, ss_b)
        drain_g(rows_a, gs_a)
        drain_s(rows_b, ss_b)
        fire_s(NGC - 1, rows_a, ss_a)
        drain_s(rows_a, ss_a)
        plsc.subcore_barrier()

        @pl.when(sid < NS - 1)
        def _():
            pltpu.sync_copy(acc_sh.at[pl.ds(r0, CRT_A)],
                            out_hbm.at[cid, pl.ds(tb + r0, CRT_A)])

        @pl.when(sid == NS - 1)
        def _():
            pltpu.sync_copy(acc_sh.at[pl.ds(r0, CRT_B)],
                            out_hbm.at[cid, pl.ds(tb + r0, CRT_B)])


BPS = B // NS
GCK = 128


@functools.partial(
    pl.kernel,
    out_type=(
        jax.ShapeDtypeStruct((NC, NT, H), jnp.float32),
        jax.ShapeDtypeStruct((NC, 3 * B, H), jnp.float32),
        jax.ShapeDtypeStruct((3 * B, WD), jnp.float32),
    ),
    mesh=_mesh,
    compiler_params=pltpu.CompilerParams(use_tc_tiling_on_sc=False),
    scratch_types=[
        pltpu.VMEM((NCHUNK_C, CKC), jnp.int32),
        pltpu.VMEM((NCHUNK_C, CKC), jnp.int32),
        pltpu.VMEM((MG, CKC, H), jnp.float32),
        pltpu.VMEM((MG, CKC, H), jnp.float32),
        pltpu.VMEM((3, BPS), jnp.int32),
        pltpu.VMEM((GCK, H), jnp.float32),
        pltpu.VMEM((GCK, WD), jnp.float32),
        pltpu.VMEM_SHARED((N, H), jnp.float32),
        pltpu.SemaphoreType.DMA,
        pltpu.SemaphoreType.DMA,
        pltpu.SemaphoreType.DMA,
        pltpu.SemaphoreType.DMA,
    ],
)
def _conv_tail_sc(src_hbm, dst_hbm, h_hbm, z_hbm, bidx_hbm, dinv_hbm,
                  out_hbm, g_hbm, dg_hbm,
                  sidx_v, didx_v, rows_a, rows_b, bidx_v, grow_v, drow_v,
                  acc_sh, gs_a, gs_b, ss_a, ss_b):
    cid = lax.axis_index("c")
    sid = lax.axis_index("s")
    wid = sid * NC + cid
    r0 = sid * CRT_A

    for t in range(3):
        tb = t * N

        @pl.when((cid == 0) & (sid < NS - 1))
        def _():
            pltpu.sync_copy(h_hbm.at[pl.ds(tb + r0, CRT_A)],
                            acc_sh.at[pl.ds(r0, CRT_A)])

        @pl.when((cid == 0) & (sid == NS - 1))
        def _():
            pltpu.sync_copy(h_hbm.at[pl.ds(tb + r0, CRT_B)],
                            acc_sh.at[pl.ds(r0, CRT_B)])

        @pl.when((cid == 1) & (sid < NS - 1))
        def _():
            pltpu.sync_copy(z_hbm, acc_sh.at[pl.ds(r0, CRT_A)])

        @pl.when((cid == 1) & (sid == NS - 1))
        def _():
            pltpu.sync_copy(z_hbm.at[pl.ds(0, CRT_B)],
                            acc_sh.at[pl.ds(r0, CRT_B)])

        pltpu.sync_copy(src_hbm.at[t, wid], sidx_v)
        pltpu.sync_copy(dst_hbm.at[t, wid], didx_v)
        plsc.subcore_barrier()

        def fire_g(g, buf, sem):
            for j in range(MG):
                pltpu.async_copy(h_hbm.at[sidx_v.at[g * MG + j]],
                                 buf.at[j], sem)

        def drain_g(buf, sem):
            for j in range(MG):
                pltpu.make_async_copy(
                    h_hbm.at[pl.ds(0, CKC)], buf.at[j], sem).wait()

        def fire_s(g, buf, sem):
            for j in range(MG):
                pltpu.async_copy(buf.at[j], acc_sh.at[didx_v.at[g * MG + j]],
                                 sem, add=True)

        def drain_s(buf, sem):
            for j in range(MG):
                pltpu.make_async_copy(
                    buf.at[j], acc_sh.at[pl.ds(0, CKC)], sem).wait()

        fire_g(0, rows_a, gs_a)
        drain_g(rows_a, gs_a)
        fire_g(1, rows_b, gs_b)
        fire_s(0, rows_a, ss_a)

        def body(i, carry):
            g = 2 * i + 1
            drain_g(rows_b, gs_b)
            drain_s(rows_a, ss_a)
            fire_g(g + 1, rows_a, gs_a)
            fire_s(g, rows_b, ss_b)
            drain_g(rows_a, gs_a)
            drain_s(rows_b, ss_b)
            fire_g(g + 2, rows_b, gs_b)
            fire_s(g + 1, rows_a, ss_a)
            return carry

        lax.fori_loop(0, (NGC - 3) // 2, body, 0)
        drain_g(rows_b, gs_b)
        drain_s(rows_a, ss_a)
        fire_g(NGC - 1, rows_a, gs_a)
        fire_s(NGC - 2, rows_b, ss_b)
        drain_g(rows_a, gs_a)
        drain_s(rows_b, ss_b)
        fire_s(NGC - 1, rows_a, ss_a)
        drain_s(rows_a, ss_a)
        plsc.subcore_barrier()

        @pl.when(sid < NS - 1)
        def _():
            pltpu.sync_copy(acc_sh.at[pl.ds(r0, CRT_A)],
                            out_hbm.at[cid, pl.ds(tb + r0, CRT_A)])

        @pl.when(sid == NS - 1)
        def _():
            pltpu.sync_copy(acc_sh.at[pl.ds(r0, CRT_B)],
                            out_hbm.at[cid, pl.ds(tb + r0, CRT_B)])

    plsc.subcore_barrier()
    pltpu.sync_copy(bidx_hbm.at[sid], bidx_v)
    for t in range(3):
        for c in range(BPS // GCK):
            pltpu.sync_copy(
                out_hbm.at[cid].at[bidx_v.at[t, pl.ds(c * GCK, GCK)]],
                grow_v)
            pltpu.sync_copy(
                grow_v,
                g_hbm.at[cid, pl.ds(t * B + sid * BPS + c * GCK, GCK)])

            @pl.when(cid == 0)
            def _():
                pltpu.sync_copy(
                    dinv_hbm.at[bidx_v.at[t, pl.ds(c * GCK, GCK)]], drow_v)
                pltpu.sync_copy(
                    drow_v,
                    dg_hbm.at[pl.ds(t * B + sid * BPS + c * GCK, GCK)])


BR = 10000
NB = NT // BR
NBT = N // BR


def _mm0_body(emb_ref, w_ref, h_ref):
    h_ref[...] = jnp.dot(
        emb_ref[...], w_ref[0], preferred_element_type=jnp.float32)


def _mm1_body(degp_ref, h0_ref, h_ref, dinv_ref):
    deg = degp_ref[0][:, 0:1] + degp_ref[1][:, 0:1] + 1.0
    dinv = lax.rsqrt(deg)
    dinv_ref[...] = jnp.broadcast_to(dinv, dinv_ref.shape)
    h_ref[...] = dinv * h0_ref[...]


def _mm2_body(p_ref, dinv_ref, b_ref, w_ref, h_ref):
    dinv = dinv_ref[:, 0:1]
    x1 = jnp.maximum(dinv * (p_ref[0] + p_ref[1]) + b_ref[0], 0.0)
    h_ref[...] = dinv * jnp.dot(x1, w_ref[0], preferred_element_type=jnp.float32)


def _head_body(g_ref, dg_ref, b2_ref, w3_ref, b3_ref, w4_ref, b4_ref, o_ref):
    def x2(t):
        sl = pl.ds(t * B, B)
        return (dg_ref[sl, 0:1] * (g_ref[0, sl] + g_ref[1, sl])
                + b2_ref[t])

    eu, el, ei = x2(0), x2(1), x2(2)
    logits = (jnp.sum(eu * ei * w3_ref[...], axis=1, keepdims=True)
              + jnp.sum(el * ei * w4_ref[...], axis=1, keepdims=True)
              + b3_ref[0, 0] + b4_ref[0, 0])
    o_ref[...] = jax.nn.sigmoid(logits)


def _matmul0(emb, w1s):
    return pl.pallas_call(
        _mm0_body,
        grid=(NB,),
        in_specs=[
            pl.BlockSpec((BR, D), lambda i: (i, 0)),
            pl.BlockSpec((1, D, H), lambda i: (i // NBT, 0, 0)),
        ],
        out_specs=pl.BlockSpec((BR, H), lambda i: (i, 0)),
        out_shape=jax.ShapeDtypeStruct((NT, H), jnp.float32),
    )(emb, w1s)


def _matmul1(degp, h0):
    return pl.pallas_call(
        _mm1_body,
        grid=(NB,),
        in_specs=[
            pl.BlockSpec((NC, BR, WD), lambda i: (0, i, 0)),
            pl.BlockSpec((BR, H), lambda i: (i, 0)),
        ],
        out_specs=[
            pl.BlockSpec((BR, H), lambda i: (i, 0)),
            pl.BlockSpec((BR, WD), lambda i: (i, 0)),
        ],
        out_shape=[
            jax.ShapeDtypeStruct((NT, H), jnp.float32),
            jax.ShapeDtypeStruct((NT, WD), jnp.float32),
        ],
    )(degp, h0)


def _matmul2(parts, dinv, b1s, w2s):
    return pl.pallas_call(
        _mm2_body,
        grid=(NB,),
        in_specs=[
            pl.BlockSpec((NC, BR, H), lambda i: (0, i, 0)),
            pl.BlockSpec((BR, WD), lambda i: (i, 0)),
            pl.BlockSpec((1, 1, H), lambda i: (i // NBT, 0, 0)),
            pl.BlockSpec((1, H, H), lambda i: (i // NBT, 0, 0)),
        ],
        out_specs=pl.BlockSpec((BR, H), lambda i: (i, 0)),
        out_shape=jax.ShapeDtypeStruct((NT, H), jnp.float32),
    )(parts, dinv, b1s, w2s)


def _head(g, dg, b2s, w3, b3, w4, b4):
    return pl.pallas_call(
        _head_body,
        grid=(1,),
        in_specs=[
            pl.BlockSpec((NC, 3 * B, H), lambda i: (0, 0, 0)),
            pl.BlockSpec((3 * B, WD), lambda i: (0, 0)),
            pl.BlockSpec((3, 1, H), lambda i: (0, 0, 0)),
            pl.BlockSpec((1, H), lambda i: (0, 0)),
            pl.BlockSpec((1, 1), lambda i: (0, 0)),
            pl.BlockSpec((1, H), lambda i: (0, 0)),
            pl.BlockSpec((1, 1), lambda i: (0, 0)),
        ],
        out_specs=pl.BlockSpec((B, 1), lambda i: (0, 0)),
        out_shape=jax.ShapeDtypeStruct((B, 1), jnp.float32),
    )(g, dg, b2s, w3, b3, w4, b4)


def kernel(user_indices, list_indices, item_indices, user_edge_index,
           list_edge_index, item_edge_index, emb_table,
           user_W1, user_b1, user_W2, user_b2,
           list_W1, list_b1, list_W2, list_b2,
           item_W1, item_b1, item_W2, item_b2,
           fc3_w, fc3_b, fc4_w, fc4_b):
    i32 = jnp.int32
    src_g = jnp.stack([
        user_edge_index[0].astype(i32),
        list_edge_index[0].astype(i32) + N,
        item_edge_index[0].astype(i32) + 2 * N,
    ]).reshape(3, NW, NCHUNK_C, CKC)
    dst_l = jnp.stack([
        user_edge_index[1].astype(i32),
        list_edge_index[1].astype(i32),
        item_edge_index[1].astype(i32),
    ]).reshape(3, NW, NCHUNK_C, CKC)
    bidx = jnp.stack([
        user_indices.astype(i32).reshape(NS, BPS),
        list_indices.astype(i32).reshape(NS, BPS) + N,
        item_indices.astype(i32).reshape(NS, BPS) + 2 * N,
    ], axis=1)
    w1s = jnp.stack([user_W1, list_W1, item_W1])
    w2s = jnp.stack([user_W2, list_W2, item_W2])
    b1s = jnp.stack([user_b1, list_b1, item_b1])[:, None, :]
    b2s = jnp.stack([user_b2, list_b2, item_b2])[:, None, :]
    dst_g = jnp.stack([
        user_edge_index[1].astype(i32),
        list_edge_index[1].astype(i32) + N,
        item_edge_index[1].astype(i32) + 2 * N,
    ]).reshape(3, NW, NCHUNK_C, CKC).swapaxes(0, 1).reshape(NW, NCHUNK, CK)

    zrows = jnp.zeros((CRT_A, H), jnp.float32)

    degp = _deg_sc(dst_g)
    h0 = _matmul0(emb_table, w1s)
    h1, dinv = _matmul1(degp, h0)
    p1 = _conv_sc(src_g, dst_l, h1, zrows)
    h2 = _matmul2(p1, dinv, b1s, w2s)
    _, g, dg = _conv_tail_sc(src_g, dst_l, h2, zrows, bidx, dinv)
    out = _head(g, dg, b2s, fc3_w.reshape(1, H), fc3_b.reshape(1, 1),
                fc4_w.reshape(1, H), fc4_b.reshape(1, 1))
    return out.reshape(-1)


# speedup vs baseline: 1.0665x; 1.0665x over previous
import functools

import jax
import jax.numpy as jnp
from jax import lax
from jax.experimental import pallas as pl
from jax.experimental.pallas import tpu as pltpu
from jax.experimental.pallas import tpu_sc as plsc

N = 10000
NT = 3 * N
E1 = 32 * N
ER = 3 * E1
D = 128
H = 64
B = 4096
WD = 16

NC, NS = 2, 16
NW = NC * NS
EPW = ER // NW
CK = 120
NCHUNK = EPW // CK
RPT_A = 1880
RPT_B = NT - 15 * RPT_A

EPW_C = E1 // NW
CKC = 80
NCHUNK_C = EPW_C // CKC
MG = 5
NGC = NCHUNK_C // MG
CRT_A = 632
CRT_B = N - 15 * CRT_A

_mesh = plsc.VectorSubcoreMesh(core_axis_name="c", subcore_axis_name="s")

DG = 10


@functools.partial(
    pl.kernel,
    out_type=jax.ShapeDtypeStruct((NC, NT, WD), jnp.float32),
    mesh=_mesh,
    compiler_params=pltpu.CompilerParams(use_tc_tiling_on_sc=False),
    scratch_types=[
        pltpu.VMEM((NCHUNK, CK), jnp.int32),
        pltpu.VMEM((CK, WD), jnp.float32),
        pltpu.VMEM((RPT_A, WD), jnp.float32),
        pltpu.VMEM_SHARED((NT, WD), jnp.float32),
        pltpu.SemaphoreType.DMA,
    ],
)
def _deg_sc(dst_hbm, out_hbm, didx_v, ones_v, zrow_v, acc_sh, ssem):
    cid = lax.axis_index("c")
    sid = lax.axis_index("s")
    wid = sid * NC + cid
    r0 = sid * RPT_A
    ones_v[...] = jnp.ones_like(ones_v)
    zrow_v[...] = jnp.zeros_like(zrow_v)

    @pl.when(sid < NS - 1)
    def _():
        pltpu.sync_copy(zrow_v, acc_sh.at[pl.ds(r0, RPT_A)])

    @pl.when(sid == NS - 1)
    def _():
        pltpu.sync_copy(zrow_v.at[pl.ds(0, RPT_B)],
                        acc_sh.at[pl.ds(r0, RPT_B)])

    pltpu.sync_copy(dst_hbm.at[wid], didx_v)
    plsc.subcore_barrier()

    def body(g, carry):
        for j in range(DG):
            pltpu.async_copy(ones_v, acc_sh.at[didx_v.at[g * DG + j]],
                             ssem, add=True)
        for j in range(DG):
            pltpu.make_async_copy(ones_v, acc_sh.at[pl.ds(0, CK)],
                                  ssem).wait()
        return carry

    lax.fori_loop(0, NCHUNK // DG, body, 0)
    plsc.subcore_barrier()

    @pl.when(sid < NS - 1)
    def _():
        pltpu.sync_copy(acc_sh.at[pl.ds(r0, RPT_A)],
                        out_hbm.at[cid, pl.ds(r0, RPT_A)])

    @pl.when(sid == NS - 1)
    def _():
        pltpu.sync_copy(acc_sh.at[pl.ds(r0, RPT_B)],
                        out_hbm.at[cid, pl.ds(r0, RPT_B)])

@functools.partial(
    pl.kernel,
    out_type=jax.ShapeDtypeStruct((NC, NT, H), jnp.float32),
    mesh=_mesh,
    compiler_params=pltpu.CompilerParams(use_tc_tiling_on_sc=False),
    scratch_types=[
        pltpu.VMEM((NCHUNK_C, CKC), jnp.int32),
        pltpu.VMEM((NCHUNK_C, CKC), jnp.int32),
        pltpu.VMEM((MG, CKC, H), jnp.float32),
        pltpu.VMEM((MG, CKC, H), jnp.float32),
        pltpu.VMEM_SHARED((N, H), jnp.float32),
        pltpu.SemaphoreType.DMA,
        pltpu.SemaphoreType.DMA,
        pltpu.SemaphoreType.DMA,
        pltpu.SemaphoreType.DMA,
        pltpu.SemaphoreType.DMA,
    ],
)
def _conv_sc(src_hbm, dst_hbm, h_hbm, z_hbm, out_hbm,
             sidx_v, didx_v, rows_a, rows_b, acc_sh,
             gs_a, gs_b, ss_a, ss_b, sd_s):
    cid = lax.axis_index("c")
    sid = lax.axis_index("s")
    wid = sid * NC + cid
    r0 = sid * CRT_A

    def fire_g(g, buf, sem):
        for j in range(MG):
            pltpu.async_copy(h_hbm.at[sidx_v.at[g * MG + j]], buf.at[j], sem)

    def drain_g(buf, sem):
        for j in range(MG):
            pltpu.make_async_copy(
                h_hbm.at[pl.ds(0, CKC)], buf.at[j], sem).wait()

    def fire_s(g, buf, sem):
        for j in range(MG):
            pltpu.async_copy(
                buf.at[j], acc_sh.at[didx_v.at[g * MG + j]], sem, add=True)

    def drain_s(buf, sem):
        for j in range(MG):
            pltpu.make_async_copy(
                buf.at[j], acc_sh.at[pl.ds(0, CKC)], sem).wait()

    for t in range(3):
        tb = t * N
        @pl.when((cid == 0) & (sid < NS - 1))
        def _():
            pltpu.async_copy(h_hbm.at[pl.ds(tb + r0, CRT_A)],
                             acc_sh.at[pl.ds(r0, CRT_A)], sd_s)

        @pl.when((cid == 0) & (sid == NS - 1))
        def _():
            pltpu.async_copy(h_hbm.at[pl.ds(tb + r0, CRT_B)],
                             acc_sh.at[pl.ds(r0, CRT_B)], sd_s)

        @pl.when((cid == 1) & (sid < NS - 1))
        def _():
            pltpu.async_copy(z_hbm, acc_sh.at[pl.ds(r0, CRT_A)], sd_s)

        @pl.when((cid == 1) & (sid == NS - 1))
        def _():
            pltpu.async_copy(z_hbm.at[pl.ds(0, CRT_B)],
                             acc_sh.at[pl.ds(r0, CRT_B)], sd_s)

        pltpu.sync_copy(src_hbm.at[t, wid], sidx_v)
        pltpu.sync_copy(dst_hbm.at[t, wid], didx_v)

        fire_g(0, rows_a, gs_a)
        drain_g(rows_a, gs_a)
        fire_g(1, rows_b, gs_b)

        @pl.when(sid < NS - 1)
        def _():
            pltpu.make_async_copy(z_hbm, acc_sh.at[pl.ds(r0, CRT_A)],
                                  sd_s).wait()

        @pl.when(sid == NS - 1)
        def _():
            pltpu.make_async_copy(z_hbm.at[pl.ds(0, CRT_B)],
                                  acc_sh.at[pl.ds(r0, CRT_B)], sd_s).wait()

        plsc.subcore_barrier()
        fire_s(0, rows_a, ss_a)

        def body(i, carry):
            g = 2 * i + 1
            drain_g(rows_b, gs_b)
            drain_s(rows_a, ss_a)
            fire_g(g + 1, rows_a, gs_a)
            fire_s(g, rows_b, ss_b)
            drain_g(rows_a, gs_a)
            drain_s(rows_b, ss_b)
            fire_g(g + 2, rows_b, gs_b)
            fire_s(g + 1, rows_a, ss_a)
            return carry

        lax.fori_loop(0, (NGC - 3) // 2, body, 0)
        drain_g(rows_b, gs_b)
        drain_s(rows_a, ss_a)
        fire_g(NGC - 1, rows_a, gs_a)
        fire_s(NGC - 2, rows_b, ss_b)
        drain_g(rows_a, gs_a)
        drain_s(rows_b, ss_b)
        fire_s(NGC - 1, rows_a, ss_a)
        drain_s(rows_a, ss_a)
        plsc.subcore_barrier()

        @pl.when(sid < NS - 1)
        def _():
            pltpu.sync_copy(acc_sh.at[pl.ds(r0, CRT_A)],
                            out_hbm.at[cid, pl.ds(tb + r0, CRT_A)])

        @pl.when(sid == NS - 1)
        def _():
            pltpu.sync_copy(acc_sh.at[pl.ds(r0, CRT_B)],
                            out_hbm.at[cid, pl.ds(tb + r0, CRT_B)])


BPS = B // NS
GCK = 128


@functools.partial(
    pl.kernel,
    out_type=(
        jax.ShapeDtypeStruct((NC, NT, H), jnp.float32),
        jax.ShapeDtypeStruct((NC, 3 * B, H), jnp.float32),
        jax.ShapeDtypeStruct((3 * B, WD), jnp.float32),
    ),
    mesh=_mesh,
    compiler_params=pltpu.CompilerParams(use_tc_tiling_on_sc=False),
    scratch_types=[
        pltpu.VMEM((NCHUNK_C, CKC), jnp.int32),
        pltpu.VMEM((NCHUNK_C, CKC), jnp.int32),
        pltpu.VMEM((MG, CKC, H), jnp.float32),
        pltpu.VMEM((MG, CKC, H), jnp.float32),
        pltpu.VMEM((3, BPS), jnp.int32),
        pltpu.VMEM((GCK, H), jnp.float32),
        pltpu.VMEM((GCK, WD), jnp.float32),
        pltpu.VMEM_SHARED((N, H), jnp.float32),
        pltpu.SemaphoreType.DMA,
        pltpu.SemaphoreType.DMA,
        pltpu.SemaphoreType.DMA,
        pltpu.SemaphoreType.DMA,
        pltpu.SemaphoreType.DMA,
    ],
)
def _conv_tail_sc(src_hbm, dst_hbm, h_hbm, z_hbm, bidx_hbm, dinv_hbm,
                  out_hbm, g_hbm, dg_hbm,
                  sidx_v, didx_v, rows_a, rows_b, bidx_v, grow_v, drow_v,
                  acc_sh, gs_a, gs_b, ss_a, ss_b, sd_s):
    cid = lax.axis_index("c")
    sid = lax.axis_index("s")
    wid = sid * NC + cid
    r0 = sid * CRT_A

    for t in range(3):
        tb = t * N

        @pl.when((cid == 0) & (sid < NS - 1))
        def _():
            pltpu.async_copy(h_hbm.at[pl.ds(tb + r0, CRT_A)],
                             acc_sh.at[pl.ds(r0, CRT_A)], sd_s)

        @pl.when((cid == 0) & (sid == NS - 1))
        def _():
            pltpu.async_copy(h_hbm.at[pl.ds(tb + r0, CRT_B)],
                             acc_sh.at[pl.ds(r0, CRT_B)], sd_s)

        @pl.when((cid == 1) & (sid < NS - 1))
        def _():
            pltpu.async_copy(z_hbm, acc_sh.at[pl.ds(r0, CRT_A)], sd_s)

        @pl.when((cid == 1) & (sid == NS - 1))
        def _():
            pltpu.async_copy(z_hbm.at[pl.ds(0, CRT_B)],
                             acc_sh.at[pl.ds(r0, CRT_B)], sd_s)

        pltpu.sync_copy(src_hbm.at[t, wid], sidx_v)
        pltpu.sync_copy(dst_hbm.at[t, wid], didx_v)

        def fire_g(g, buf, sem):
            for j in range(MG):
                pltpu.async_copy(h_hbm.at[sidx_v.at[g * MG + j]],
                                 buf.at[j], sem)

        def drain_g(buf, sem):
            for j in range(MG):
                pltpu.make_async_copy(
                    h_hbm.at[pl.ds(0, CKC)], buf.at[j], sem).wait()

        def fire_s(g, buf, sem):
            for j in range(MG):
                pltpu.async_copy(buf.at[j], acc_sh.at[didx_v.at[g * MG + j]],
                                 sem, add=True)

        def drain_s(buf, sem):
            for j in range(MG):
                pltpu.make_async_copy(
                    buf.at[j], acc_sh.at[pl.ds(0, CKC)], sem).wait()

        fire_g(0, rows_a, gs_a)
        drain_g(rows_a, gs_a)
        fire_g(1, rows_b, gs_b)

        @pl.when(sid < NS - 1)
        def _():
            pltpu.make_async_copy(z_hbm, acc_sh.at[pl.ds(r0, CRT_A)],
                                  sd_s).wait()

        @pl.when(sid == NS - 1)
        def _():
            pltpu.make_async_copy(z_hbm.at[pl.ds(0, CRT_B)],
                                  acc_sh.at[pl.ds(r0, CRT_B)], sd_s).wait()

        plsc.subcore_barrier()
        fire_s(0, rows_a, ss_a)

        def body(i, carry):
            g = 2 * i + 1
            drain_g(rows_b, gs_b)
            drain_s(rows_a, ss_a)
            fire_g(g + 1, rows_a, gs_a)
            fire_s(g, rows_b, ss_b)
            drain_g(rows_a, gs_a)
            drain_s(rows_b, ss_b)
            fire_g(g + 2, rows_b, gs_b)
            fire_s(g + 1, rows_a, ss_a)
            return carry

        lax.fori_loop(0, (NGC - 3) // 2, body, 0)
        drain_g(rows_b, gs_b)
        drain_s(rows_a, ss_a)
        fire_g(NGC - 1, rows_a, gs_a)
        fire_s(NGC - 2, rows_b, ss_b)
        drain_g(rows_a, gs_a)
        drain_s(rows_b, ss_b)
        fire_s(NGC - 1, rows_a, ss_a)
        drain_s(rows_a, ss_a)
        plsc.subcore_barrier()

        @pl.when(sid < NS - 1)
        def _():
            pltpu.sync_copy(acc_sh.at[pl.ds(r0, CRT_A)],
                            out_hbm.at[cid, pl.ds(tb + r0, CRT_A)])

        @pl.when(sid == NS - 1)
        def _():
            pltpu.sync_copy(acc_sh.at[pl.ds(r0, CRT_B)],
                            out_hbm.at[cid, pl.ds(tb + r0, CRT_B)])

    plsc.subcore_barrier()
    pltpu.sync_copy(bidx_hbm.at[sid], bidx_v)
    for t in range(3):
        for c in range(BPS // GCK):
            pltpu.sync_copy(
                out_hbm.at[cid].at[bidx_v.at[t, pl.ds(c * GCK, GCK)]],
                grow_v)
            pltpu.sync_copy(
                grow_v,
                g_hbm.at[cid, pl.ds(t * B + sid * BPS + c * GCK, GCK)])

            @pl.when(cid == 0)
            def _():
                pltpu.sync_copy(
                    dinv_hbm.at[bidx_v.at[t, pl.ds(c * GCK, GCK)]], drow_v)
                pltpu.sync_copy(
                    drow_v,
                    dg_hbm.at[pl.ds(t * B + sid * BPS + c * GCK, GCK)])


BR = 10000
NB = NT // BR
NBT = N // BR


def _mm1_body(degp_ref, emb_ref, w_ref, h_ref, dinv_ref):
    deg = degp_ref[0][:, 0:1] + degp_ref[1][:, 0:1] + 1.0
    dinv = lax.rsqrt(deg)
    dinv_ref[...] = jnp.broadcast_to(dinv, dinv_ref.shape)
    h_ref[...] = dinv * jnp.dot(
        emb_ref[...], w_ref[0], preferred_element_type=jnp.float32)


def _mm2_body(p_ref, dinv_ref, b_ref, w_ref, h_ref):
    dinv = dinv_ref[:, 0:1]
    x1 = jnp.maximum(dinv * (p_ref[0] + p_ref[1]) + b_ref[0], 0.0)
    h_ref[...] = dinv * jnp.dot(x1, w_ref[0], preferred_element_type=jnp.float32)


def _head_body(g_ref, dg_ref, b2_ref, w3_ref, b3_ref, w4_ref, b4_ref, o_ref):
    def x2(t):
        sl = pl.ds(t * B, B)
        return (dg_ref[sl, 0:1] * (g_ref[0, sl] + g_ref[1, sl])
                + b2_ref[t])

    eu, el, ei = x2(0), x2(1), x2(2)
    logits = (jnp.sum(eu * ei * w3_ref[...], axis=1, keepdims=True)
              + jnp.sum(el * ei * w4_ref[...], axis=1, keepdims=True)
              + b3_ref[0, 0] + b4_ref[0, 0])
    o_ref[...] = jax.nn.sigmoid(logits)


def _matmul1(degp, emb, w1s):
    return pl.pallas_call(
        _mm1_body,
        grid=(NB,),
        in_specs=[
            pl.BlockSpec((NC, BR, WD), lambda i: (0, i, 0)),
            pl.BlockSpec((BR, D), lambda i: (i, 0)),
            pl.BlockSpec((1, D, H), lambda i: (i // NBT, 0, 0)),
        ],
        out_specs=[
            pl.BlockSpec((BR, H), lambda i: (i, 0)),
            pl.BlockSpec((BR, WD), lambda i: (i, 0)),
        ],
        out_shape=[
            jax.ShapeDtypeStruct((NT, H), jnp.float32),
            jax.ShapeDtypeStruct((NT, WD), jnp.float32),
        ],
    )(degp, emb, w1s)


def _matmul2(parts, dinv, b1s, w2s):
    return pl.pallas_call(
        _mm2_body,
        grid=(NB,),
        in_specs=[
            pl.BlockSpec((NC, BR, H), lambda i: (0, i, 0)),
            pl.BlockSpec((BR, WD), lambda i: (i, 0)),
            pl.BlockSpec((1, 1, H), lambda i: (i // NBT, 0, 0)),
            pl.BlockSpec((1, H, H), lambda i: (i // NBT, 0, 0)),
        ],
        out_specs=pl.BlockSpec((BR, H), lambda i: (i, 0)),
        out_shape=jax.ShapeDtypeStruct((NT, H), jnp.float32),
    )(parts, dinv, b1s, w2s)


def _head(g, dg, b2s, w3, b3, w4, b4):
    return pl.pallas_call(
        _head_body,
        grid=(1,),
        in_specs=[
            pl.BlockSpec((NC, 3 * B, H), lambda i: (0, 0, 0)),
            pl.BlockSpec((3 * B, WD), lambda i: (0, 0)),
            pl.BlockSpec((3, 1, H), lambda i: (0, 0, 0)),
            pl.BlockSpec((1, H), lambda i: (0, 0)),
            pl.BlockSpec((1, 1), lambda i: (0, 0)),
            pl.BlockSpec((1, H), lambda i: (0, 0)),
            pl.BlockSpec((1, 1), lambda i: (0, 0)),
        ],
        out_specs=pl.BlockSpec((B, 1), lambda i: (0, 0)),
        out_shape=jax.ShapeDtypeStruct((B, 1), jnp.float32),
    )(g, dg, b2s, w3, b3, w4, b4)


def kernel(user_indices, list_indices, item_indices, user_edge_index,
           list_edge_index, item_edge_index, emb_table,
           user_W1, user_b1, user_W2, user_b2,
           list_W1, list_b1, list_W2, list_b2,
           item_W1, item_b1, item_W2, item_b2,
           fc3_w, fc3_b, fc4_w, fc4_b):
    i32 = jnp.int32
    src_g = jnp.stack([
        user_edge_index[0].astype(i32),
        list_edge_index[0].astype(i32) + N,
        item_edge_index[0].astype(i32) + 2 * N,
    ]).reshape(3, NW, NCHUNK_C, CKC)
    dst_l = jnp.stack([
        user_edge_index[1].astype(i32),
        list_edge_index[1].astype(i32),
        item_edge_index[1].astype(i32),
    ]).reshape(3, NW, NCHUNK_C, CKC)
    bidx = jnp.stack([
        user_indices.astype(i32).reshape(NS, BPS),
        list_indices.astype(i32).reshape(NS, BPS) + N,
        item_indices.astype(i32).reshape(NS, BPS) + 2 * N,
    ], axis=1)
    w1s = jnp.stack([user_W1, list_W1, item_W1])
    w2s = jnp.stack([user_W2, list_W2, item_W2])
    b1s = jnp.stack([user_b1, list_b1, item_b1])[:, None, :]
    b2s = jnp.stack([user_b2, list_b2, item_b2])[:, None, :]
    dst_g = jnp.concatenate([
        user_edge_index[1].astype(i32),
        list_edge_index[1].astype(i32) + N,
        item_edge_index[1].astype(i32) + 2 * N,
    ]).reshape(NW, NCHUNK, CK)

    zrows = jnp.zeros((CRT_A, H), jnp.float32)

    degp = _deg_sc(dst_g)
    h1, dinv = _matmul1(degp, emb_table, w1s)
    p1 = _conv_sc(src_g, dst_l, h1, zrows)
    h2 = _matmul2(p1, dinv, b1s, w2s)
    _, g, dg = _conv_tail_sc(src_g, dst_l, h2, zrows, bidx, dinv)
    out = _head(g, dg, b2s, fc3_w.reshape(1, H), fc3_b.reshape(1, 1),
                fc4_w.reshape(1, H), fc4_b.reshape(1, 1))
    return out.reshape(-1)
